# baseline (speedup 1.0000x reference)
import jax
import jax.numpy as jnp
from jax.experimental import pallas as pl


def _gcn_conv(x, edge_index, W, b):
    n = x.shape[0]
    loop = jnp.arange(n, dtype=edge_index.dtype)
    src = jnp.concatenate([edge_index[0], loop])
    dst = jnp.concatenate([edge_index[1], loop])
    deg = jnp.zeros((n,), dtype=x.dtype).at[dst].add(1.0)
    dis = jnp.where(deg > 0, jax.lax.rsqrt(jnp.maximum(deg, 1e-12)), 0.0)
    norm = dis[src] * dis[dst]
    xw = x @ W
    out = jnp.zeros((n, W.shape[1]), dtype=x.dtype).at[dst].add(xw[src] * norm[:, None])
    return out + b


def _batch_norm(h, gamma, beta, eps=1e-5):
    mean = jnp.mean(h, axis=0)
    var = jnp.var(h, axis=0)
    return (h - mean) * jax.lax.rsqrt(var + eps) * gamma + beta


def _copy_kernel(a_ref, o_ref):
    o_ref[...] = a_ref[...]


def _pl_copy(a):
    return pl.pallas_call(
        _copy_kernel,
        out_shape=jax.ShapeDtypeStruct(a.shape, a.dtype),
    )(a)


def kernel(x, edge_index, W1, b1, g1, be1, W2, b2, g2, be2, Wmu, bmu, Wls, bls):
    h = _gcn_conv(x, edge_index, W1, b1)
    h = jax.nn.relu(_batch_norm(h, g1, be1))
    h = _gcn_conv(h, edge_index, W2, b2)
    h = jax.nn.relu(_batch_norm(h, g2, be2))
    mu = _gcn_conv(h, edge_index, Wmu, bmu)
    logstd = _gcn_conv(h, edge_index, Wls, bls)
    return (_pl_copy(mu), _pl_copy(logstd))


# speedup vs baseline: 20.2632x; 20.2632x over previous
import functools

import jax
import jax.numpy as jnp
from jax import lax
from jax.experimental import pallas as pl
from jax.experimental.pallas import tpu as pltpu
from jax.experimental.pallas import tpu_sc as plsc

NC = 2
NS = 16
LANES = 16
CHUNK = 128
NBUF = 4
ROWS = 1000
EPS = 1e-5


def _sc_degree(edges_pad, zeros8, ones8, n_acc):
    e_pad = edges_pad.shape[1]
    e_per_tile = e_pad // (NC * NS)
    n_chunks = e_per_tile // CHUNK
    nrt = n_acc // NS
    mesh = plsc.VectorSubcoreMesh(core_axis_name="c", subcore_axis_name="s")

    @functools.partial(
        pl.kernel,
        out_type=jax.ShapeDtypeStruct((NC, n_acc, 8), jnp.float32),
        mesh=mesh,
        compiler_params=pltpu.CompilerParams(use_tc_tiling_on_sc=False),
        scratch_types=[
            pltpu.VMEM((NBUF, CHUNK), jnp.int32),
            pltpu.VMEM((CHUNK, 8), jnp.float32),
            pltpu.VMEM_SHARED((n_acc, 8), jnp.float32),
        ] + [pltpu.SemaphoreType.DMA] * NBUF,
    )
    def deg_kernel(edges_hbm, zeros_hbm, ones_hbm, out_hbm, didx, ones_v,
                   acc, *ssem):
        c = lax.axis_index("c")
        s = lax.axis_index("s")
        pltpu.sync_copy(zeros_hbm.at[pl.ds(s * nrt, nrt)],
                        acc.at[pl.ds(s * nrt, nrt)])
        pltpu.sync_copy(ones_hbm, ones_v)
        plsc.subcore_barrier()
        base = (c * NS + s) * e_per_tile

        def idx_load(j, b):
            pltpu.sync_copy(edges_hbm.at[1, pl.ds(base + j * CHUNK, CHUNK)],
                            didx.at[b])

        def scatter_start(b):
            return pltpu.async_copy(ones_v, acc.at[didx.at[b]], ssem[b],
                                    add=True)

        def scatter_wait(b):
            pltpu.make_async_copy(ones_v, acc.at[didx.at[b]], ssem[b]).wait()

        for b in range(NBUF):
            idx_load(b, b)

        @pl.loop(NBUF, n_chunks, step=NBUF)
        def _(j0):
            for b in range(NBUF):
                scatter_start(b)
            for b in range(NBUF):
                scatter_wait(b)
                idx_load(j0 + b, b)

        for b in range(NBUF):
            scatter_start(b)
        for b in range(NBUF):
            scatter_wait(b)

        plsc.subcore_barrier()
        pltpu.sync_copy(acc.at[pl.ds(s * nrt, nrt)],
                        out_hbm.at[c, pl.ds(s * nrt, nrt)])

    return deg_kernel(edges_pad, zeros8, ones8)


def _sc_aggregate(xs_flat, edges_pad, n_acc, half):
    e_pad = edges_pad.shape[1]
    e_per_tile = e_pad // NS
    n_chunks = e_per_tile // CHUNK
    nrt = n_acc // NS
    mesh = plsc.VectorSubcoreMesh(core_axis_name="c", subcore_axis_name="s")

    @functools.partial(
        pl.kernel,
        out_type=jax.ShapeDtypeStruct((NC, n_acc, half), jnp.float32),
        mesh=mesh,
        compiler_params=pltpu.CompilerParams(use_tc_tiling_on_sc=False),
        scratch_types=[
            pltpu.VMEM((NBUF, 2, CHUNK), jnp.int32),
            pltpu.VMEM((NBUF, CHUNK, half), jnp.float32),
            pltpu.VMEM_SHARED((n_acc, half), jnp.float32),
        ] + [pltpu.SemaphoreType.DMA] * (2 * NBUF),
    )
    def agg_kernel(xs_hbm, edges_hbm, out_hbm, eidx, rows, acc, *sems):
        gsem, ssem = sems[:NBUF], sems[NBUF:]
        c = lax.axis_index("c")
        s = lax.axis_index("s")
        coff = c * n_acc
        pltpu.sync_copy(xs_hbm.at[pl.ds(coff + s * nrt, nrt)],
                        acc.at[pl.ds(s * nrt, nrt)])
        plsc.subcore_barrier()
        base = s * e_per_tile

        def load_start(j, b):
            pltpu.sync_copy(edges_hbm.at[:, pl.ds(base + j * CHUNK, CHUNK)],
                            eidx.at[b])
            for v in range(CHUNK // LANES):
                sl = pl.ds(v * LANES, LANES)
                eidx[b, 0, sl] = eidx[b, 0, sl] + coff
            return pltpu.async_copy(xs_hbm.at[eidx.at[b, 0]], rows.at[b],
                                    gsem[b])

        def gather_wait(b):
            pltpu.make_async_copy(xs_hbm.at[eidx.at[b, 0]], rows.at[b],
                                  gsem[b]).wait()

        def scatter_start(b):
            return pltpu.async_copy(rows.at[b], acc.at[eidx.at[b, 1]],
                                    ssem[b], add=True)

        def scatter_wait(b):
            pltpu.make_async_copy(rows.at[b], acc.at[eidx.at[b, 1]],
                                  ssem[b]).wait()

        for b in range(NBUF):
            load_start(b, b)

        @pl.loop(NBUF, n_chunks, step=NBUF)
        def _(j0):
            for b in range(NBUF):
                gather_wait(b)
                scatter_start(b)
            for b in range(NBUF):
                scatter_wait(b)
                load_start(j0 + b, b)

        for b in range(NBUF):
            gather_wait(b)
            scatter_start(b)
        for b in range(NBUF):
            scatter_wait(b)

        plsc.subcore_barrier()
        pltpu.sync_copy(acc.at[pl.ds(s * nrt, nrt)],
                        out_hbm.at[c, pl.ds(s * nrt, nrt)])

    return agg_kernel(xs_flat, edges_pad)


def _prep_body(x_ref, w_ref, degp_ref, xs_ref, dis_ref):
    degp = degp_ref[...]
    deg = degp[0, :, 0] + degp[1, :, 0] + 1.0
    dis = lax.rsqrt(deg)
    xw = jnp.dot(x_ref[...], w_ref[...], preferred_element_type=jnp.float32)
    xs = xw * dis[:, None]
    half = xs.shape[1] // 2
    xs_ref[0] = xs[:, :half]
    xs_ref[1] = xs[:, half:]
    dis_ref[...] = dis[:, None]


def _bn_stats_body(n_rows, acc_ref, dis_ref, b_ref, z_ref, st_ref, ssum):
    i = pl.program_id(0)

    @pl.when(i == 0)
    def _():
        ssum[...] = jnp.zeros_like(ssum)

    dis = dis_ref[...][:, 0]
    z = acc_ref[...] * dis[None, :, None] + b_ref[...][:, None, :]
    z_ref[...] = z
    ssum[0] += jnp.sum(z, axis=1)
    ssum[1] += jnp.sum(z * z, axis=1)
    st_ref[...] = ssum[...]
    del n_rows


def _bn_apply_mm_body(n_rows, z_ref, st_ref, g_ref, be_ref, dis_ref, w_ref,
                      out_ref):
    st = st_ref[...]
    mean = st[0] / n_rows
    var = st[1] / n_rows - mean * mean
    scale = lax.rsqrt(var + EPS) * g_ref[...]
    shift = be_ref[...] - mean * scale
    h = jnp.maximum(z_ref[...] * scale[:, None, :] + shift[:, None, :], 0.0)
    hf = jnp.concatenate([h[0], h[1]], axis=1)
    xw = jnp.dot(hf, w_ref[...], preferred_element_type=jnp.float32)
    xs = xw * dis_ref[...]
    half = xw.shape[1] // 2
    out_ref[0] = xs[:, :half]
    out_ref[1] = xs[:, half:]


def _bn_apply_body(n_rows, z_ref, st_ref, g_ref, be_ref, dis_ref, out_ref):
    st = st_ref[...]
    mean = st[0] / n_rows
    var = st[1] / n_rows - mean * mean
    scale = lax.rsqrt(var + EPS) * g_ref[...]
    shift = be_ref[...] - mean * scale
    h = jnp.maximum(z_ref[...] * scale[:, None, :] + shift[:, None, :], 0.0)
    out_ref[...] = h * dis_ref[...][None, :, :]


def _final_body(acc_ref, dis_ref, wmu_ref, bmu_ref, wls_ref, bls_ref,
                mu_ref, ls_ref):
    dis = dis_ref[...]
    a = acc_ref[...]
    t = jnp.concatenate([a[0], a[1]], axis=1) * dis
    mu_ref[...] = (jnp.dot(t, wmu_ref[...], preferred_element_type=jnp.float32)
                   + bmu_ref[...])
    ls_ref[...] = (jnp.dot(t, wls_ref[...], preferred_element_type=jnp.float32)
                   + bls_ref[...])


def _full_spec(shape):
    zeros = (0,) * len(shape)
    return pl.BlockSpec(shape, lambda i: zeros)


def _row_spec(nd_shape):
    if len(nd_shape) == 3:
        return pl.BlockSpec((nd_shape[0], ROWS, nd_shape[2]),
                            lambda i: (0, i, 0))
    return pl.BlockSpec((ROWS, nd_shape[1]), lambda i: (i, 0))


def kernel(x, edge_index, W1, b1, g1, be1, W2, b2, g2, be2, Wmu, bmu, Wls,
           bls):
    n, in_ch = x.shape
    hid = W1.shape[1]
    lat = Wmu.shape[1]
    half = hid // 2
    e = edge_index.shape[1]
    idt = edge_index.dtype

    n_blocks = n // ROWS
    nrt = -(-(n + 1) // (NS * 8)) * 8
    n_acc = nrt * NS

    e_unit = NC * NS * CHUNK * NBUF
    e_pad = -(-e // e_unit) * e_unit
    pad = e_pad - e
    pad_col = jnp.concatenate(
        [jnp.zeros((1, pad), idt), jnp.full((1, pad), n, idt)])
    edges_p = jnp.concatenate([edge_index, pad_col], axis=1)

    degp = _sc_degree(edges_p, jnp.zeros((n_acc, 8), jnp.float32),
                      jnp.ones((CHUNK, 8), jnp.float32), n_acc)

    xs1, dis = pl.pallas_call(
        _prep_body,
        grid=(n_blocks,),
        in_specs=[_row_spec((n, in_ch)), _full_spec(W1.shape),
                  _row_spec((NC, n_acc, 8))],
        out_specs=[_row_spec((NC, n_acc, half)), _row_spec((n_acc, 1))],
        out_shape=[jax.ShapeDtypeStruct((NC, n_acc, half), jnp.float32),
                   jax.ShapeDtypeStruct((n_acc, 1), jnp.float32)],
    )(x, W1, degp)

    nf = float(n)

    def bn_stats(acc, b):
        return pl.pallas_call(
            functools.partial(_bn_stats_body, nf),
            grid=(n_blocks,),
            in_specs=[_row_spec((NC, n_acc, half)), _row_spec((n_acc, 1)),
                      _full_spec((NC, half))],
            out_specs=[_row_spec((NC, n_acc, half)),
                       _full_spec((2, NC, half))],
            out_shape=[jax.ShapeDtypeStruct((NC, n_acc, half), jnp.float32),
                       jax.ShapeDtypeStruct((2, NC, half), jnp.float32)],
            scratch_shapes=[pltpu.VMEM((2, NC, half), jnp.float32)],
        )(acc, dis, b.reshape(NC, half))

    acc1 = _sc_aggregate(xs1.reshape(NC * n_acc, half), edges_p, n_acc, half)
    z1, st1 = bn_stats(acc1, b1)
    xs2 = pl.pallas_call(
        functools.partial(_bn_apply_mm_body, nf),
        grid=(n_blocks,),
        in_specs=[_row_spec((NC, n_acc, half)), _full_spec((2, NC, half)),
                  _full_spec((NC, half)), _full_spec((NC, half)),
                  _row_spec((n_acc, 1)), _full_spec(W2.shape)],
        out_specs=_row_spec((NC, n_acc, half)),
        out_shape=jax.ShapeDtypeStruct((NC, n_acc, half), jnp.float32),
    )(z1, st1, g1.reshape(NC, half), be1.reshape(NC, half), dis, W2)

    acc2 = _sc_aggregate(xs2.reshape(NC * n_acc, half), edges_p, n_acc, half)
    z2, st2 = bn_stats(acc2, b2)
    xs3 = pl.pallas_call(
        functools.partial(_bn_apply_body, nf),
        grid=(n_blocks,),
        in_specs=[_row_spec((NC, n_acc, half)), _full_spec((2, NC, half)),
                  _full_spec((NC, half)), _full_spec((NC, half)),
                  _row_spec((n_acc, 1))],
        out_specs=_row_spec((NC, n_acc, half)),
        out_shape=jax.ShapeDtypeStruct((NC, n_acc, half), jnp.float32),
    )(z2, st2, g2.reshape(NC, half), be2.reshape(NC, half), dis)

    acc3 = _sc_aggregate(xs3.reshape(NC * n_acc, half), edges_p, n_acc, half)
    mu, ls = pl.pallas_call(
        _final_body,
        grid=(n_blocks,),
        in_specs=[_row_spec((NC, n_acc, half)), _row_spec((n_acc, 1)),
                  _full_spec(Wmu.shape), _full_spec((1, lat)),
                  _full_spec(Wls.shape), _full_spec((1, lat))],
        out_specs=[_row_spec((n, lat)), _row_spec((n, lat))],
        out_shape=[jax.ShapeDtypeStruct((n, lat), jnp.float32),
                   jax.ShapeDtypeStruct((n, lat), jnp.float32)],
    )(acc3, dis, Wmu, bmu.reshape(1, lat), Wls, bls.reshape(1, lat))

    return (mu, ls)
